# baseline (speedup 1.0000x reference)
import functools

import jax
import jax.numpy as jnp
from jax import lax
from jax.experimental import pallas as pl
from jax.experimental.pallas import tpu as pltpu
from jax.experimental.pallas import tpu_sc as plsc

N = 10000
E = 320000
D = 128
NC = 2
NS = 16
NW = NC * NS
EPW = E // NW
C = 128
BLK = 8
EPWP = 10240
NCH = EPWP // C
NBLK = NCH // BLK
NPAD = 10112
RPT = NPAD // NS

_mesh = plsc.VectorSubcoreMesh(core_axis_name="c", subcore_axis_name="s")


@functools.partial(
    pl.kernel,
    out_type=(
        jax.ShapeDtypeStruct((NC, NPAD, D), jnp.float32),
        jax.ShapeDtypeStruct((NW, NCH, C), jnp.int32),
    ),
    mesh=_mesh,
    scratch_types=[
        pltpu.VMEM((BLK, C), jnp.int32),
        pltpu.VMEM((BLK, C), jnp.int32),
        pltpu.VMEM((BLK, C), jnp.int32),
        pltpu.VMEM((C, D), jnp.float32),
        pltpu.VMEM_SHARED((NPAD, D), jnp.float32),
    ],
)
def _deg_kernel(row_hbm, col_hbm, ones_hbm, zeros_hbm, degacc_hbm, colm_hbm,
                row_v, col_v, rowm_v, ones_v, acc):
    cid = lax.axis_index("c")
    sid = lax.axis_index("s")
    wid = sid * NC + cid
    slab = pl.ds(sid * RPT, RPT)

    pltpu.sync_copy(ones_hbm, ones_v)
    pltpu.sync_copy(zeros_hbm.at[slab], acc.at[slab])
    plsc.subcore_barrier()

    dummy = jnp.full((16,), N, jnp.int32)

    def block_body(b, carry):
        bs = pl.ds(b * BLK, BLK)
        pltpu.sync_copy(row_hbm.at[wid, bs], row_v)
        pltpu.sync_copy(col_hbm.at[wid, bs], col_v)
        for i in range(BLK):
            for g in range(C // 16):
                s = pl.ds(g * 16, 16)
                r = row_v[i, s]
                c = col_v[i, s]
                m = (r == c) | (c == dummy)
                rowm_v[i, s] = jnp.where(m, dummy, r)
                col_v[i, s] = jnp.where(m, dummy, c)
        pltpu.sync_copy(col_v, colm_hbm.at[wid, bs])
        for q in range(BLK):
            pltpu.sync_copy(ones_v, acc.at[rowm_v.at[q]], add=True)
        return carry

    lax.fori_loop(0, NBLK, block_body, 0, unroll=False)

    plsc.subcore_barrier()
    pltpu.sync_copy(acc.at[slab], degacc_hbm.at[cid].at[slab])


@functools.partial(
    pl.kernel,
    out_type=jax.ShapeDtypeStruct((NC, NPAD, D), jnp.float32),
    mesh=_mesh,
    scratch_types=[
        pltpu.VMEM((2, BLK, C), jnp.int32),
        pltpu.VMEM((2, BLK, C), jnp.int32),
        pltpu.VMEM((C, D), jnp.float32),
        pltpu.VMEM((C, D), jnp.float32),
        pltpu.SemaphoreType.DMA,
        pltpu.SemaphoreType.DMA,
        pltpu.SemaphoreType.DMA,
        pltpu.SemaphoreType.DMA,
        pltpu.VMEM_SHARED((NPAD, D), jnp.float32),
    ],
)
def _prop_kernel(xs_hbm, row_hbm, colm_hbm, zeros_hbm, out_hbm,
                 row_v, colm_v, buf0, buf1, sem0, sem1, sem2, sem3, acc):
    cid = lax.axis_index("c")
    sid = lax.axis_index("s")
    wid = sid * NC + cid
    slab = pl.ds(sid * RPT, RPT)

    pltpu.sync_copy(zeros_hbm.at[slab], acc.at[slab])
    plsc.subcore_barrier()

    pltpu.sync_copy(row_hbm.at[wid, pl.ds(0, BLK)], row_v.at[0])
    pltpu.sync_copy(colm_hbm.at[wid, pl.ds(0, BLK)], colm_v.at[0])
    bufs = (buf0, buf1)
    sems = ((sem0, sem2), (sem1, sem3))
    H = C // 2

    def _issue(idx_row, k):
        pltpu.async_copy(xs_hbm.at[idx_row.at[pl.ds(0, H)]],
                         bufs[k].at[pl.ds(0, H)], sems[k][0])
        pltpu.async_copy(xs_hbm.at[idx_row.at[pl.ds(H, H)]],
                         bufs[k].at[pl.ds(H, H)], sems[k][1])

    def _drain(idx_row, k):
        pltpu.make_async_copy(xs_hbm.at[idx_row.at[pl.ds(0, H)]],
                              bufs[k].at[pl.ds(0, H)], sems[k][0]).wait()
        pltpu.make_async_copy(xs_hbm.at[idx_row.at[pl.ds(H, H)]],
                              bufs[k].at[pl.ds(H, H)], sems[k][1]).wait()

    _issue(row_v.at[0, 0], 0)
    _issue(row_v.at[0, 1], 1)

    def super_body(t, carry):
        for p in range(2):
            b = 2 * t + p

            @pl.when(b + 1 < NBLK)
            def _():
                nbs = pl.ds((b + 1) * BLK, BLK)
                pltpu.sync_copy(row_hbm.at[wid, nbs], row_v.at[1 - p])
                pltpu.sync_copy(colm_hbm.at[wid, nbs], colm_v.at[1 - p])

            for q in range(BLK):
                _drain(row_v.at[p, q], q % 2)
                pltpu.sync_copy(bufs[q % 2], acc.at[colm_v.at[p, q]],
                                add=True)
                if q + 2 < BLK:
                    _issue(row_v.at[p, q + 2], q % 2)
                else:

                    @pl.when(b + 1 < NBLK)
                    def _():
                        _issue(row_v.at[1 - p, q + 2 - BLK], q % 2)
        return carry

    lax.fori_loop(0, NBLK // 2, super_body, 0, unroll=False)

    plsc.subcore_barrier()
    pltpu.sync_copy(acc.at[slab], out_hbm.at[cid].at[slab])


def _bn_relu(y, g, b):
    mean = jnp.mean(y, axis=0, keepdims=True)
    var = jnp.mean((y - mean) ** 2, axis=0, keepdims=True)
    return jnp.maximum((y - mean) * lax.rsqrt(var + 1e-5) * g + b, 0.0)


def _mm(a, w):
    return jnp.dot(a, w, preferred_element_type=jnp.float32)


def _prep_body(degacc, x, w10, b1, g1, be1, w20, dis_o, h1_o, xs1_o, y2_o):
    deg = degacc[0, :N, 0:1] + degacc[1, :N, 0:1]
    dis = jnp.where(deg > 0, lax.rsqrt(jnp.maximum(deg, 1e-12)), 0.0)
    h = _bn_relu(_mm(x[...], w10[...]) + b1[...], g1[...], be1[...])
    dis_o[...] = dis
    h1_o[...] = h
    xs1_o[...] = dis * h
    y2_o[...] = _mm(h, w20[...])


def _mid_body(S, dis, wk, y_in, t_o, xs_o, y_o):
    t = -dis[...] * (S[0, :N, :] + S[1, :N, :])
    t_o[...] = t
    xs_o[...] = dis[...] * t
    y_o[...] = y_in[...] + _mm(t, wk[...])


def _mid2_body(S, dis, wk, y_in, tx0, t_o, xs_o, y_o):
    t = -2.0 * dis[...] * (S[0, :N, :] + S[1, :N, :]) - tx0[...]
    t_o[...] = t
    xs_o[...] = dis[...] * t
    y_o[...] = y_in[...] + _mm(t, wk[...])


def _end_body(S, dis, tx0, wk, y_in, b2, g2, be2, wn0, h_o, xs_o, yn_o):
    t = -2.0 * dis[...] * (S[0, :N, :] + S[1, :N, :]) - tx0[...]
    y = y_in[...] + _mm(t, wk[...]) + b2[...]
    h = _bn_relu(y, g2[...], be2[...])
    h_o[...] = h
    xs_o[...] = dis[...] * h
    yn_o[...] = _mm(h, wn0[...])


def _final_body(S, dis, tx0, wk, y_in, b3, out_o):
    t = -2.0 * dis[...] * (S[0, :N, :] + S[1, :N, :]) - tx0[...]
    out_o[...] = y_in[...] + _mm(t, wk[...]) + b3[...]


_nd = jax.ShapeDtypeStruct((N, D), jnp.float32)
_tc_prep = pl.pallas_call(_prep_body, out_shape=(
    jax.ShapeDtypeStruct((N, 1), jnp.float32), _nd, _nd, _nd))
_tc_mid = pl.pallas_call(_mid_body, out_shape=(_nd, _nd, _nd))
_tc_mid2 = pl.pallas_call(_mid2_body, out_shape=(_nd, _nd, _nd))
_tc_end = pl.pallas_call(_end_body, out_shape=(_nd, _nd, _nd))
_tc_final = pl.pallas_call(_final_body, out_shape=_nd)


def kernel(x, edge_index, W1, b1, W2, b2, W3, b3, gamma1, beta1, gamma2,
           beta2):
    pad = EPWP - EPW
    row = jnp.pad(edge_index[0].reshape(NW, EPW), ((0, 0), (0, pad)),
                  constant_values=0).reshape(NW, NCH, C)
    col = jnp.pad(edge_index[1].reshape(NW, EPW), ((0, 0), (0, pad)),
                  constant_values=N).reshape(NW, NCH, C)
    zeros = jnp.zeros((NPAD, D), jnp.float32)
    ones = jnp.ones((C, D), jnp.float32)
    b1r = b1.reshape(1, D)
    b2r = b2.reshape(1, D)
    b3r = b3.reshape(1, D)
    g1 = gamma1.reshape(1, D)
    be1 = beta1.reshape(1, D)
    g2 = gamma2.reshape(1, D)
    be2 = beta2.reshape(1, D)

    degacc, colm = _deg_kernel(row, col, ones, zeros)

    dis, h1, xs1, y2a = _tc_prep(degacc, x, W1[0], b1r, g1, be1, W2[0])

    S1 = _prop_kernel(xs1, row, colm, zeros)
    t1, xs_t1, y2b = _tc_mid(S1, dis, W2[1], y2a)
    S2 = _prop_kernel(xs_t1, row, colm, zeros)
    h2, xs2, y3a = _tc_end(S2, dis, h1, W2[2], y2b, b2r, g2, be2, W3[0])

    S3 = _prop_kernel(xs2, row, colm, zeros)
    u1, xs_u1, y3b = _tc_mid(S3, dis, W3[1], y3a)
    S4 = _prop_kernel(xs_u1, row, colm, zeros)
    u2, xs_u2, y3c = _tc_mid2(S4, dis, W3[2], y3b, h2)
    S5 = _prop_kernel(xs_u2, row, colm, zeros)
    u3, xs_u3, y3d = _tc_mid2(S5, dis, W3[3], y3c, u1)
    S6 = _prop_kernel(xs_u3, row, colm, zeros)
    return _tc_final(S6, dis, u2, W3[4], y3d, b3r)


# speedup vs baseline: 2.8794x; 2.8794x over previous
import functools

import jax
import jax.numpy as jnp
from jax import lax
from jax.experimental import pallas as pl
from jax.experimental.pallas import tpu as pltpu
from jax.experimental.pallas import tpu_sc as plsc

N = 10000
E = 320000
D = 128
NC = 2
NS = 16
NW = NC * NS
EPW = E // NW
C = 128
BLK = 8
EPWP = 10240
NCH = EPWP // C
NBLK = NCH // BLK
NPAD = 10112
RPT = NPAD // NS

_mesh = plsc.VectorSubcoreMesh(core_axis_name="c", subcore_axis_name="s")


@functools.partial(
    pl.kernel,
    out_type=(
        jax.ShapeDtypeStruct((NC, NPAD, D), jnp.float32),
        jax.ShapeDtypeStruct((NW, NCH, C), jnp.int32),
    ),
    mesh=_mesh,
    scratch_types=[
        pltpu.VMEM((BLK, C), jnp.int32),
        pltpu.VMEM((BLK, C), jnp.int32),
        pltpu.VMEM((BLK, C), jnp.int32),
        pltpu.VMEM((C, D), jnp.float32),
        pltpu.VMEM_SHARED((NPAD, D), jnp.float32),
    ],
)
def _deg_kernel(row_hbm, col_hbm, ones_hbm, zeros_hbm, degacc_hbm, colm_hbm,
                row_v, col_v, rowm_v, ones_v, acc):
    cid = lax.axis_index("c")
    sid = lax.axis_index("s")
    wid = sid * NC + cid
    slab = pl.ds(sid * RPT, RPT)

    pltpu.sync_copy(ones_hbm, ones_v)
    pltpu.sync_copy(zeros_hbm.at[slab], acc.at[slab])
    plsc.subcore_barrier()

    def block_body(b, carry):
        bs = pl.ds(b * BLK, BLK)
        pltpu.sync_copy(row_hbm.at[wid, bs], row_v)
        pltpu.sync_copy(col_hbm.at[wid, bs], col_v)
        for i in range(BLK):
            for g in range(C // 16):
                s = pl.ds(g * 16, 16)
                dummy = N + ((lax.iota(jnp.int32, 16) + 16 * g) & 63)
                r = row_v[i, s]
                c = col_v[i, s]
                m = (r == c) | (c >= N)
                rowm_v[i, s] = jnp.where(m, dummy, r)
                col_v[i, s] = jnp.where(m, dummy, c)
        pltpu.sync_copy(col_v, colm_hbm.at[wid, bs])
        for q in range(BLK):
            pltpu.sync_copy(ones_v, acc.at[rowm_v.at[q]], add=True)
        return carry

    lax.fori_loop(0, NBLK, block_body, 0, unroll=False)

    plsc.subcore_barrier()
    pltpu.sync_copy(acc.at[slab], degacc_hbm.at[cid].at[slab])


@functools.partial(
    pl.kernel,
    out_type=jax.ShapeDtypeStruct((NC, NPAD, D), jnp.float32),
    mesh=_mesh,
    scratch_types=[
        pltpu.VMEM((2, BLK, C), jnp.int32),
        pltpu.VMEM((2, BLK, C), jnp.int32),
        pltpu.VMEM((C, D), jnp.float32),
        pltpu.VMEM((C, D), jnp.float32),
        pltpu.SemaphoreType.DMA,
        pltpu.SemaphoreType.DMA,
        pltpu.VMEM_SHARED((NPAD, D), jnp.float32),
    ],
)
def _prop_kernel(xs_hbm, row_hbm, colm_hbm, zeros_hbm, out_hbm,
                 row_v, colm_v, buf0, buf1, sem0, sem1, acc):
    cid = lax.axis_index("c")
    sid = lax.axis_index("s")
    wid = sid * NC + cid
    slab = pl.ds(sid * RPT, RPT)

    pltpu.sync_copy(zeros_hbm.at[slab], acc.at[slab])
    plsc.subcore_barrier()

    pltpu.sync_copy(row_hbm.at[wid, pl.ds(0, BLK)], row_v.at[0])
    pltpu.sync_copy(colm_hbm.at[wid, pl.ds(0, BLK)], colm_v.at[0])
    bufs = (buf0, buf1)
    sems = (sem0, sem1)

    def _issue(idx_row, k):
        pltpu.async_copy(xs_hbm.at[idx_row], bufs[k], sems[k])

    def _drain(idx_row, k):
        pltpu.make_async_copy(xs_hbm.at[idx_row], bufs[k], sems[k]).wait()

    _issue(row_v.at[0, 0], 0)
    _issue(row_v.at[0, 1], 1)

    def super_body(t, carry):
        for p in range(2):
            b = 2 * t + p

            @pl.when(b + 1 < NBLK)
            def _():
                nbs = pl.ds((b + 1) * BLK, BLK)
                pltpu.sync_copy(row_hbm.at[wid, nbs], row_v.at[1 - p])
                pltpu.sync_copy(colm_hbm.at[wid, nbs], colm_v.at[1 - p])

            for q in range(BLK):
                _drain(row_v.at[p, q], q % 2)
                pltpu.sync_copy(bufs[q % 2], acc.at[colm_v.at[p, q]],
                                add=True)
                if q + 2 < BLK:
                    _issue(row_v.at[p, q + 2], q % 2)
                else:

                    @pl.when(b + 1 < NBLK)
                    def _():
                        _issue(row_v.at[1 - p, q + 2 - BLK], q % 2)
        return carry

    lax.fori_loop(0, NBLK // 2, super_body, 0, unroll=False)

    plsc.subcore_barrier()
    pltpu.sync_copy(acc.at[slab], out_hbm.at[cid].at[slab])


def _bn_relu(y, g, b):
    mean = jnp.mean(y, axis=0, keepdims=True)
    var = jnp.mean((y - mean) ** 2, axis=0, keepdims=True)
    return jnp.maximum((y - mean) * lax.rsqrt(var + 1e-5) * g + b, 0.0)


def _mm(a, w):
    return jnp.dot(a, w, preferred_element_type=jnp.float32)


def _prep_body(degacc, x, w10, b1, g1, be1, w20, dis_o, h1_o, xs1_o, y2_o):
    deg = degacc[0, :N, 0:1] + degacc[1, :N, 0:1]
    dis = jnp.where(deg > 0, lax.rsqrt(jnp.maximum(deg, 1e-12)), 0.0)
    h = _bn_relu(_mm(x[...], w10[...]) + b1[...], g1[...], be1[...])
    dis_o[...] = dis
    h1_o[...] = h
    xs1_o[...] = dis * h
    y2_o[...] = _mm(h, w20[...])


def _mid_body(S, dis, wk, y_in, t_o, xs_o, y_o):
    t = -dis[...] * (S[0, :N, :] + S[1, :N, :])
    t_o[...] = t
    xs_o[...] = dis[...] * t
    y_o[...] = y_in[...] + _mm(t, wk[...])


def _mid2_body(S, dis, wk, y_in, tx0, t_o, xs_o, y_o):
    t = -2.0 * dis[...] * (S[0, :N, :] + S[1, :N, :]) - tx0[...]
    t_o[...] = t
    xs_o[...] = dis[...] * t
    y_o[...] = y_in[...] + _mm(t, wk[...])


def _end_body(S, dis, tx0, wk, y_in, b2, g2, be2, wn0, h_o, xs_o, yn_o):
    t = -2.0 * dis[...] * (S[0, :N, :] + S[1, :N, :]) - tx0[...]
    y = y_in[...] + _mm(t, wk[...]) + b2[...]
    h = _bn_relu(y, g2[...], be2[...])
    h_o[...] = h
    xs_o[...] = dis[...] * h
    yn_o[...] = _mm(h, wn0[...])


def _final_body(S, dis, tx0, wk, y_in, b3, out_o):
    t = -2.0 * dis[...] * (S[0, :N, :] + S[1, :N, :]) - tx0[...]
    out_o[...] = y_in[...] + _mm(t, wk[...]) + b3[...]


_nd = jax.ShapeDtypeStruct((N, D), jnp.float32)
_tc_prep = pl.pallas_call(_prep_body, out_shape=(
    jax.ShapeDtypeStruct((N, 1), jnp.float32), _nd, _nd, _nd))
_tc_mid = pl.pallas_call(_mid_body, out_shape=(_nd, _nd, _nd))
_tc_mid2 = pl.pallas_call(_mid2_body, out_shape=(_nd, _nd, _nd))
_tc_end = pl.pallas_call(_end_body, out_shape=(_nd, _nd, _nd))
_tc_final = pl.pallas_call(_final_body, out_shape=_nd)


def kernel(x, edge_index, W1, b1, W2, b2, W3, b3, gamma1, beta1, gamma2,
           beta2):
    pad = EPWP - EPW
    padrows = jnp.broadcast_to(jnp.arange(pad, dtype=jnp.int32) % 128,
                               (NW, pad))
    row = jnp.concatenate(
        [edge_index[0].reshape(NW, EPW), padrows], axis=1).reshape(NW, NCH, C)
    col = jnp.pad(edge_index[1].reshape(NW, EPW), ((0, 0), (0, pad)),
                  constant_values=N).reshape(NW, NCH, C)
    zeros = jnp.zeros((NPAD, D), jnp.float32)
    ones = jnp.ones((C, D), jnp.float32)
    b1r = b1.reshape(1, D)
    b2r = b2.reshape(1, D)
    b3r = b3.reshape(1, D)
    g1 = gamma1.reshape(1, D)
    be1 = beta1.reshape(1, D)
    g2 = gamma2.reshape(1, D)
    be2 = beta2.reshape(1, D)

    degacc, colm = _deg_kernel(row, col, ones, zeros)

    dis, h1, xs1, y2a = _tc_prep(degacc, x, W1[0], b1r, g1, be1, W2[0])

    S1 = _prop_kernel(xs1, row, colm, zeros)
    t1, xs_t1, y2b = _tc_mid(S1, dis, W2[1], y2a)
    S2 = _prop_kernel(xs_t1, row, colm, zeros)
    h2, xs2, y3a = _tc_end(S2, dis, h1, W2[2], y2b, b2r, g2, be2, W3[0])

    S3 = _prop_kernel(xs2, row, colm, zeros)
    u1, xs_u1, y3b = _tc_mid(S3, dis, W3[1], y3a)
    S4 = _prop_kernel(xs_u1, row, colm, zeros)
    u2, xs_u2, y3c = _tc_mid2(S4, dis, W3[2], y3b, h2)
    S5 = _prop_kernel(xs_u2, row, colm, zeros)
    u3, xs_u3, y3d = _tc_mid2(S5, dis, W3[3], y3c, u1)
    S6 = _prop_kernel(xs_u3, row, colm, zeros)
    return _tc_final(S6, dis, u2, W3[4], y3d, b3r)


# speedup vs baseline: 2.8983x; 1.0066x over previous
import functools

import jax
import jax.numpy as jnp
from jax import lax
from jax.experimental import pallas as pl
from jax.experimental.pallas import tpu as pltpu
from jax.experimental.pallas import tpu_sc as plsc

N = 10000
E = 320000
D = 128
NC = 2
NS = 16
NW = NC * NS
EPW = E // NW
C = 128
BLK = 8
EPWP = 10240
NCH = EPWP // C
NBLK = NCH // BLK
NPAD = 10112
RPT = NPAD // NS

_mesh = plsc.VectorSubcoreMesh(core_axis_name="c", subcore_axis_name="s")


@functools.partial(
    pl.kernel,
    out_type=(
        jax.ShapeDtypeStruct((NC, NPAD, D), jnp.float32),
        jax.ShapeDtypeStruct((NW, NCH, C), jnp.int32),
    ),
    mesh=_mesh,
    scratch_types=[
        pltpu.VMEM((BLK, C), jnp.int32),
        pltpu.VMEM((BLK, C), jnp.int32),
        pltpu.VMEM((BLK, C), jnp.int32),
        pltpu.VMEM((C, D), jnp.float32),
        pltpu.VMEM_SHARED((NPAD, D), jnp.float32),
    ],
)
def _deg_kernel(row_hbm, col_hbm, ones_hbm, zeros_hbm, degacc_hbm, colm_hbm,
                row_v, col_v, rowm_v, ones_v, acc):
    cid = lax.axis_index("c")
    sid = lax.axis_index("s")
    wid = sid * NC + cid
    slab = pl.ds(sid * RPT, RPT)

    pltpu.sync_copy(ones_hbm, ones_v)
    pltpu.sync_copy(zeros_hbm.at[slab], acc.at[slab])
    plsc.subcore_barrier()

    def block_body(b, carry):
        bs = pl.ds(b * BLK, BLK)
        pltpu.sync_copy(row_hbm.at[wid, bs], row_v)
        pltpu.sync_copy(col_hbm.at[wid, bs], col_v)
        for i in range(BLK):
            for g in range(C // 16):
                s = pl.ds(g * 16, 16)
                dummy = N + ((lax.iota(jnp.int32, 16) + 16 * g) & 63)
                r = row_v[i, s]
                c = col_v[i, s]
                m = (r == c) | (c >= N)
                rowm_v[i, s] = jnp.where(m, dummy, r)
                col_v[i, s] = jnp.where(m, dummy, c)
        pltpu.sync_copy(col_v, colm_hbm.at[wid, bs])
        for q in range(BLK):
            pltpu.sync_copy(ones_v, acc.at[rowm_v.at[q]], add=True)
        return carry

    lax.fori_loop(0, NBLK, block_body, 0, unroll=False)

    plsc.subcore_barrier()
    pltpu.sync_copy(acc.at[slab], degacc_hbm.at[cid].at[slab])


@functools.partial(
    pl.kernel,
    out_type=jax.ShapeDtypeStruct((NC, NPAD, D), jnp.float32),
    mesh=_mesh,
    scratch_types=[
        pltpu.VMEM((2, BLK, C), jnp.int32),
        pltpu.VMEM((2, BLK, C), jnp.int32),
        pltpu.VMEM((C, D), jnp.float32),
        pltpu.VMEM((C, D), jnp.float32),
        pltpu.SemaphoreType.DMA,
        pltpu.SemaphoreType.DMA,
        pltpu.VMEM_SHARED((NPAD, D), jnp.float32),
    ],
)
def _prop_kernel(xs_hbm, row_hbm, colm_hbm, zeros_hbm, out_hbm,
                 row_v, colm_v, buf0, buf1, sem0, sem1, acc):
    cid = lax.axis_index("c")
    sid = lax.axis_index("s")
    wid = sid * NC + cid
    slab = pl.ds(sid * RPT, RPT)

    pltpu.sync_copy(zeros_hbm.at[slab], acc.at[slab])
    plsc.subcore_barrier()

    pltpu.sync_copy(row_hbm.at[wid, pl.ds(0, BLK)], row_v.at[0])
    pltpu.sync_copy(colm_hbm.at[wid, pl.ds(0, BLK)], colm_v.at[0])
    bufs = (buf0, buf1)
    sems = (sem0, sem1)

    def _issue(idx_row, k):
        pltpu.async_copy(xs_hbm.at[idx_row], bufs[k], sems[k])

    def _drain(idx_row, k):
        pltpu.make_async_copy(xs_hbm.at[idx_row], bufs[k], sems[k]).wait()

    _issue(row_v.at[0, 0], 0)
    _issue(row_v.at[0, 1], 1)

    def super_body(t, carry):
        for p in range(2):
            b = 2 * t + p

            @pl.when(b + 1 < NBLK)
            def _():
                nbs = pl.ds((b + 1) * BLK, BLK)
                pltpu.sync_copy(row_hbm.at[wid, nbs], row_v.at[1 - p])
                pltpu.sync_copy(colm_hbm.at[wid, nbs], colm_v.at[1 - p])

            for q in range(BLK):
                _drain(row_v.at[p, q], q % 2)
                pltpu.sync_copy(bufs[q % 2], acc.at[colm_v.at[p, q]],
                                add=True)
                if q + 2 < BLK:
                    _issue(row_v.at[p, q + 2], q % 2)
                else:

                    @pl.when(b + 1 < NBLK)
                    def _():
                        _issue(row_v.at[1 - p, q + 2 - BLK], q % 2)
        return carry

    lax.fori_loop(0, NBLK // 2, super_body, 0, unroll=False)

    plsc.subcore_barrier()
    pltpu.sync_copy(acc.at[slab], out_hbm.at[cid].at[slab])


def _bn_relu(y, g, b):
    mean = jnp.mean(y, axis=0, keepdims=True)
    var = jnp.mean((y - mean) ** 2, axis=0, keepdims=True)
    return jnp.maximum((y - mean) * lax.rsqrt(var + 1e-5) * g + b, 0.0)


def _mm(a, w):
    return jnp.dot(a, w, preferred_element_type=jnp.float32)


def _prep_body(degacc, x, w10, b1, g1, be1, w20, dis_o, h1_o, xs1_o, y2_o):
    deg = degacc[0, :N, 0:1] + degacc[1, :N, 0:1]
    dis = jnp.where(deg > 0, lax.rsqrt(jnp.maximum(deg, 1e-12)), 0.0)
    h = _bn_relu(_mm(x[...], w10[...]) + b1[...], g1[...], be1[...])
    dis_o[...] = dis
    h1_o[...] = h
    xs1_o[...] = dis * h
    y2_o[...] = _mm(h, w20[...])


def _txs_body(S, dis, t_o, xs_o):
    t = -dis[...] * (S[0, :N, :] + S[1, :N, :])
    t_o[...] = t
    xs_o[...] = dis[...] * t


def _txs2_body(S, dis, tx0, t_o, xs_o):
    t = -2.0 * dis[...] * (S[0, :N, :] + S[1, :N, :]) - tx0[...]
    t_o[...] = t
    xs_o[...] = dis[...] * t


def _acc_body(y_in, t, wk, y_o):
    y_o[...] = y_in[...] + _mm(t[...], wk[...])


def _end_body(S, dis, tx0, wk, y_in, b2, g2, be2, wn0, h_o, xs_o, yn_o):
    t = -2.0 * dis[...] * (S[0, :N, :] + S[1, :N, :]) - tx0[...]
    y = y_in[...] + _mm(t, wk[...]) + b2[...]
    h = _bn_relu(y, g2[...], be2[...])
    h_o[...] = h
    xs_o[...] = dis[...] * h
    yn_o[...] = _mm(h, wn0[...])


def _final_body(S, dis, tx0, wk, y_in, b3, out_o):
    t = -2.0 * dis[...] * (S[0, :N, :] + S[1, :N, :]) - tx0[...]
    out_o[...] = y_in[...] + _mm(t, wk[...]) + b3[...]


_nd = jax.ShapeDtypeStruct((N, D), jnp.float32)
_tc_prep = pl.pallas_call(_prep_body, out_shape=(
    jax.ShapeDtypeStruct((N, 1), jnp.float32), _nd, _nd, _nd))
_tc_txs = pl.pallas_call(_txs_body, out_shape=(_nd, _nd))
_tc_txs2 = pl.pallas_call(_txs2_body, out_shape=(_nd, _nd))
_tc_acc = pl.pallas_call(_acc_body, out_shape=_nd)
_tc_end = pl.pallas_call(_end_body, out_shape=(_nd, _nd, _nd))
_tc_final = pl.pallas_call(_final_body, out_shape=_nd)


def kernel(x, edge_index, W1, b1, W2, b2, W3, b3, gamma1, beta1, gamma2,
           beta2):
    pad = EPWP - EPW
    padrows = jnp.broadcast_to(jnp.arange(pad, dtype=jnp.int32) % 128,
                               (NW, pad))
    row = jnp.concatenate(
        [edge_index[0].reshape(NW, EPW), padrows], axis=1).reshape(NW, NCH, C)
    col = jnp.pad(edge_index[1].reshape(NW, EPW), ((0, 0), (0, pad)),
                  constant_values=N).reshape(NW, NCH, C)
    zeros = jnp.zeros((NPAD, D), jnp.float32)
    ones = jnp.ones((C, D), jnp.float32)
    b1r = b1.reshape(1, D)
    b2r = b2.reshape(1, D)
    b3r = b3.reshape(1, D)
    g1 = gamma1.reshape(1, D)
    be1 = beta1.reshape(1, D)
    g2 = gamma2.reshape(1, D)
    be2 = beta2.reshape(1, D)

    degacc, colm = _deg_kernel(row, col, ones, zeros)

    dis, h1, xs1, y2a = _tc_prep(degacc, x, W1[0], b1r, g1, be1, W2[0])

    S1 = _prop_kernel(xs1, row, colm, zeros)
    t1, xs_t1 = _tc_txs(S1, dis)
    S2 = _prop_kernel(xs_t1, row, colm, zeros)
    y2b = _tc_acc(y2a, t1, W2[1])
    h2, xs2, y3a = _tc_end(S2, dis, h1, W2[2], y2b, b2r, g2, be2, W3[0])

    S3 = _prop_kernel(xs2, row, colm, zeros)
    u1, xs_u1 = _tc_txs(S3, dis)
    S4 = _prop_kernel(xs_u1, row, colm, zeros)
    y3b = _tc_acc(y3a, u1, W3[1])
    u2, xs_u2 = _tc_txs2(S4, dis, h2)
    S5 = _prop_kernel(xs_u2, row, colm, zeros)
    y3c = _tc_acc(y3b, u2, W3[2])
    u3, xs_u3 = _tc_txs2(S5, dis, u1)
    S6 = _prop_kernel(xs_u3, row, colm, zeros)
    y3d = _tc_acc(y3c, u3, W3[3])
    return _tc_final(S6, dis, u2, W3[4], y3d, b3r)


# speedup vs baseline: 2.9680x; 1.0240x over previous
import functools

import jax
import jax.numpy as jnp
from jax import lax
from jax.experimental import pallas as pl
from jax.experimental.pallas import tpu as pltpu
from jax.experimental.pallas import tpu_sc as plsc

N = 10000
E = 320000
D = 128
NC = 2
NS = 16
NW = NC * NS
EPW = E // NW
C = 128
BLK = 8
EPWP = 10240
NCH = EPWP // C
NBLK = NCH // BLK
NPAD = 10112
RPT = NPAD // NS

_mesh = plsc.VectorSubcoreMesh(core_axis_name="c", subcore_axis_name="s")


@functools.partial(
    pl.kernel,
    out_type=(
        jax.ShapeDtypeStruct((NC, NPAD, D), jnp.float32),
        jax.ShapeDtypeStruct((NW, NCH, 2, C), jnp.int32),
    ),
    mesh=_mesh,
    scratch_types=[
        pltpu.VMEM((BLK, C), jnp.int32),
        pltpu.VMEM((BLK, C), jnp.int32),
        pltpu.VMEM((BLK, C), jnp.int32),
        pltpu.VMEM((C, D), jnp.float32),
        pltpu.VMEM_SHARED((NPAD, D), jnp.float32),
    ],
)
def _deg_kernel(row_hbm, col_hbm, ones_hbm, zeros_hbm, degacc_hbm, colm_hbm,
                row_v, col_v, rowm_v, ones_v, acc):
    cid = lax.axis_index("c")
    sid = lax.axis_index("s")
    wid = sid * NC + cid
    slab = pl.ds(sid * RPT, RPT)

    pltpu.sync_copy(ones_hbm, ones_v)
    pltpu.sync_copy(zeros_hbm.at[slab], acc.at[slab])
    plsc.subcore_barrier()

    def block_body(b, carry):
        bs = pl.ds(b * BLK, BLK)
        pltpu.sync_copy(row_hbm.at[wid, bs], row_v)
        pltpu.sync_copy(col_hbm.at[wid, bs], col_v)
        for i in range(BLK):
            for g in range(C // 16):
                s = pl.ds(g * 16, 16)
                dummy = N + ((lax.iota(jnp.int32, 16) + 16 * g) & 63)
                r = row_v[i, s]
                c = col_v[i, s]
                m = (r == c) | (c >= N)
                rowm_v[i, s] = jnp.where(m, dummy, r)
                col_v[i, s] = jnp.where(m, dummy, c)
        pltpu.sync_copy(row_v, colm_hbm.at[wid, bs, 0])
        pltpu.sync_copy(col_v, colm_hbm.at[wid, bs, 1])
        for q in range(BLK):
            pltpu.sync_copy(ones_v, acc.at[rowm_v.at[q]], add=True)
        return carry

    lax.fori_loop(0, NBLK, block_body, 0, unroll=False)

    plsc.subcore_barrier()
    pltpu.sync_copy(acc.at[slab], degacc_hbm.at[cid].at[slab])


@functools.partial(
    pl.kernel,
    out_type=jax.ShapeDtypeStruct((NC, NPAD, D), jnp.float32),
    mesh=_mesh,
    scratch_types=[
        pltpu.VMEM((2, BLK, 2, C), jnp.int32),
        pltpu.VMEM((C, D), jnp.float32),
        pltpu.VMEM((C, D), jnp.float32),
        pltpu.SemaphoreType.DMA,
        pltpu.SemaphoreType.DMA,
        pltpu.VMEM_SHARED((NPAD, D), jnp.float32),
    ],
)
def _prop_kernel(xs_hbm, ric_hbm, zeros_hbm, out_hbm,
                 ric_v, buf0, buf1, sem0, sem1, acc):
    cid = lax.axis_index("c")
    sid = lax.axis_index("s")
    wid = sid * NC + cid
    slab = pl.ds(sid * RPT, RPT)

    pltpu.sync_copy(zeros_hbm.at[slab], acc.at[slab])
    plsc.subcore_barrier()

    pltpu.sync_copy(ric_hbm.at[wid, pl.ds(0, BLK)], ric_v.at[0])
    bufs = (buf0, buf1)
    sems = (sem0, sem1)

    def _issue(idx_row, k):
        pltpu.async_copy(xs_hbm.at[idx_row], bufs[k], sems[k])

    def _drain(idx_row, k):
        pltpu.make_async_copy(xs_hbm.at[idx_row], bufs[k], sems[k]).wait()

    _issue(ric_v.at[0, 0, 0], 0)
    _issue(ric_v.at[0, 1, 0], 1)

    def super_body(t, carry):
        for p in range(2):
            b = 2 * t + p

            @pl.when(b + 1 < NBLK)
            def _():
                nbs = pl.ds((b + 1) * BLK, BLK)
                pltpu.sync_copy(ric_hbm.at[wid, nbs], ric_v.at[1 - p])

            for q in range(BLK):
                _drain(ric_v.at[p, q, 0], q % 2)
                pltpu.sync_copy(bufs[q % 2], acc.at[ric_v.at[p, q, 1]],
                                add=True)
                if q + 2 < BLK:
                    _issue(ric_v.at[p, q + 2, 0], q % 2)
                else:

                    @pl.when(b + 1 < NBLK)
                    def _():
                        _issue(ric_v.at[1 - p, q + 2 - BLK, 0], q % 2)
        return carry

    lax.fori_loop(0, NBLK // 2, super_body, 0, unroll=False)

    plsc.subcore_barrier()
    pltpu.sync_copy(acc.at[slab], out_hbm.at[cid].at[slab])


def _bn_relu(y, g, b):
    mean = jnp.mean(y, axis=0, keepdims=True)
    var = jnp.mean((y - mean) ** 2, axis=0, keepdims=True)
    return jnp.maximum((y - mean) * lax.rsqrt(var + 1e-5) * g + b, 0.0)


def _mm(a, w):
    return jnp.dot(a, w, preferred_element_type=jnp.float32)


def _prep_body(degacc, x, w10, b1, g1, be1, w20, dis_o, h1_o, xs1_o, y2_o):
    deg = degacc[0, :N, 0:1] + degacc[1, :N, 0:1]
    dis = jnp.where(deg > 0, lax.rsqrt(jnp.maximum(deg, 1e-12)), 0.0)
    h = _bn_relu(_mm(x[...], w10[...]) + b1[...], g1[...], be1[...])
    dis_o[...] = dis
    h1_o[...] = h
    xs1_o[...] = dis * h
    y2_o[...] = _mm(h, w20[...])


def _txs_body(S, dis, t_o, xs_o):
    t = -dis[...] * (S[0, :N, :] + S[1, :N, :])
    t_o[...] = t
    xs_o[...] = dis[...] * t


def _txs2_body(S, dis, tx0, t_o, xs_o):
    t = -2.0 * dis[...] * (S[0, :N, :] + S[1, :N, :]) - tx0[...]
    t_o[...] = t
    xs_o[...] = dis[...] * t


def _acc_body(y_in, t, wk, y_o):
    y_o[...] = y_in[...] + _mm(t[...], wk[...])


def _end_body(S, dis, tx0, wk, y_in, b2, g2, be2, wn0, h_o, xs_o, yn_o):
    t = -2.0 * dis[...] * (S[0, :N, :] + S[1, :N, :]) - tx0[...]
    y = y_in[...] + _mm(t, wk[...]) + b2[...]
    h = _bn_relu(y, g2[...], be2[...])
    h_o[...] = h
    xs_o[...] = dis[...] * h
    yn_o[...] = _mm(h, wn0[...])


def _final_body(S, dis, tx0, wk, y_in, b3, out_o):
    t = -2.0 * dis[...] * (S[0, :N, :] + S[1, :N, :]) - tx0[...]
    out_o[...] = y_in[...] + _mm(t, wk[...]) + b3[...]


_nd = jax.ShapeDtypeStruct((N, D), jnp.float32)
_tc_prep = pl.pallas_call(_prep_body, out_shape=(
    jax.ShapeDtypeStruct((N, 1), jnp.float32), _nd, _nd, _nd))
_tc_txs = pl.pallas_call(_txs_body, out_shape=(_nd, _nd))
_tc_txs2 = pl.pallas_call(_txs2_body, out_shape=(_nd, _nd))
_tc_acc = pl.pallas_call(_acc_body, out_shape=_nd)
_tc_end = pl.pallas_call(_end_body, out_shape=(_nd, _nd, _nd))
_tc_final = pl.pallas_call(_final_body, out_shape=_nd)


def kernel(x, edge_index, W1, b1, W2, b2, W3, b3, gamma1, beta1, gamma2,
           beta2):
    pad = EPWP - EPW
    padrows = jnp.broadcast_to(jnp.arange(pad, dtype=jnp.int32) % 128,
                               (NW, pad))
    row = jnp.concatenate(
        [edge_index[0].reshape(NW, EPW), padrows], axis=1).reshape(NW, NCH, C)
    col = jnp.pad(edge_index[1].reshape(NW, EPW), ((0, 0), (0, pad)),
                  constant_values=N).reshape(NW, NCH, C)
    zeros = jnp.zeros((NPAD, D), jnp.float32)
    ones = jnp.ones((C, D), jnp.float32)
    b1r = b1.reshape(1, D)
    b2r = b2.reshape(1, D)
    b3r = b3.reshape(1, D)
    g1 = gamma1.reshape(1, D)
    be1 = beta1.reshape(1, D)
    g2 = gamma2.reshape(1, D)
    be2 = beta2.reshape(1, D)

    degacc, colm = _deg_kernel(row, col, ones, zeros)

    dis, h1, xs1, y2a = _tc_prep(degacc, x, W1[0], b1r, g1, be1, W2[0])

    S1 = _prop_kernel(xs1, colm, zeros)
    t1, xs_t1 = _tc_txs(S1, dis)
    S2 = _prop_kernel(xs_t1, colm, zeros)
    y2b = _tc_acc(y2a, t1, W2[1])
    h2, xs2, y3a = _tc_end(S2, dis, h1, W2[2], y2b, b2r, g2, be2, W3[0])

    S3 = _prop_kernel(xs2, colm, zeros)
    u1, xs_u1 = _tc_txs(S3, dis)
    S4 = _prop_kernel(xs_u1, colm, zeros)
    y3b = _tc_acc(y3a, u1, W3[1])
    u2, xs_u2 = _tc_txs2(S4, dis, h2)
    S5 = _prop_kernel(xs_u2, colm, zeros)
    y3c = _tc_acc(y3b, u2, W3[2])
    u3, xs_u3 = _tc_txs2(S5, dis, u1)
    S6 = _prop_kernel(xs_u3, colm, zeros)
    y3d = _tc_acc(y3c, u3, W3[3])
    return _tc_final(S6, dis, u2, W3[4], y3d, b3r)


# speedup vs baseline: 3.0137x; 1.0154x over previous
import functools

import jax
import jax.numpy as jnp
from jax import lax
from jax.experimental import pallas as pl
from jax.experimental.pallas import tpu as pltpu
from jax.experimental.pallas import tpu_sc as plsc

N = 10000
E = 320000
D = 128
NC = 2
NS = 16
NW = NC * NS
EPW = E // NW
C = 128
BLK = 8
EPWP = 10240
NCH = EPWP // C
NBLK = NCH // BLK
NPAD = 10112
RPT = NPAD // NS

_mesh = plsc.VectorSubcoreMesh(core_axis_name="c", subcore_axis_name="s")


@functools.partial(
    pl.kernel,
    out_type=(
        jax.ShapeDtypeStruct((NC, NPAD, D), jnp.float32),
        jax.ShapeDtypeStruct((NW, NCH, 2, C), jnp.int32),
    ),
    mesh=_mesh,
    scratch_types=[
        pltpu.VMEM((BLK, C), jnp.int32),
        pltpu.VMEM((BLK, C), jnp.int32),
        pltpu.VMEM((BLK, C), jnp.int32),
        pltpu.VMEM((C, D), jnp.float32),
        pltpu.VMEM_SHARED((NPAD, D), jnp.float32),
    ],
)
def _deg_kernel(row_hbm, col_hbm, ones_hbm, zeros_hbm, degacc_hbm, colm_hbm,
                row_v, col_v, rowm_v, ones_v, acc):
    cid = lax.axis_index("c")
    sid = lax.axis_index("s")
    wid = sid * NC + cid
    slab = pl.ds(sid * RPT, RPT)

    pltpu.sync_copy(ones_hbm, ones_v)
    pltpu.sync_copy(zeros_hbm.at[slab], acc.at[slab])
    plsc.subcore_barrier()

    def block_body(b, carry):
        bs = pl.ds(b * BLK, BLK)
        pltpu.sync_copy(row_hbm.at[wid, bs], row_v)
        pltpu.sync_copy(col_hbm.at[wid, bs], col_v)
        for i in range(BLK):
            for g in range(C // 16):
                s = pl.ds(g * 16, 16)
                dummy = N + ((lax.iota(jnp.int32, 16) + 16 * g) & 63)
                r = row_v[i, s]
                c = col_v[i, s]
                m = (r == c) | (c >= N)
                rowm_v[i, s] = jnp.where(m, dummy, r)
                col_v[i, s] = jnp.where(m, dummy, c)
        pltpu.sync_copy(row_v, colm_hbm.at[wid, bs, 0])
        pltpu.sync_copy(col_v, colm_hbm.at[wid, bs, 1])
        for q in range(BLK):
            pltpu.sync_copy(ones_v, acc.at[rowm_v.at[q]], add=True)
        return carry

    lax.fori_loop(0, NBLK, block_body, 0, unroll=False)

    plsc.subcore_barrier()
    pltpu.sync_copy(acc.at[slab], degacc_hbm.at[cid].at[slab])


@functools.partial(
    pl.kernel,
    out_type=jax.ShapeDtypeStruct((NC, NPAD, D), jnp.float32),
    mesh=_mesh,
    scratch_types=[
        pltpu.VMEM((2, BLK, 2, C), jnp.int32),
        pltpu.VMEM((C, D), jnp.float32),
        pltpu.VMEM((C, D), jnp.float32),
        pltpu.SemaphoreType.DMA,
        pltpu.SemaphoreType.DMA,
        pltpu.SemaphoreType.DMA,
        pltpu.VMEM_SHARED((NPAD, D), jnp.float32),
    ],
)
def _prop_kernel(xs_hbm, ric_hbm, zeros_hbm, out_hbm,
                 ric_v, buf0, buf1, sem0, sem1, semb, acc):
    cid = lax.axis_index("c")
    sid = lax.axis_index("s")
    wid = sid * NC + cid
    slab = pl.ds(sid * RPT, RPT)

    pltpu.sync_copy(zeros_hbm.at[slab], acc.at[slab])
    plsc.subcore_barrier()

    pltpu.sync_copy(ric_hbm.at[wid, pl.ds(0, BLK)], ric_v.at[0])
    pltpu.async_copy(ric_hbm.at[wid, pl.ds(BLK, BLK)], ric_v.at[1], semb)
    bufs = (buf0, buf1)
    sems = (sem0, sem1)

    def _issue(idx_row, k):
        pltpu.async_copy(xs_hbm.at[idx_row], bufs[k], sems[k])

    def _drain(idx_row, k):
        pltpu.make_async_copy(xs_hbm.at[idx_row], bufs[k], sems[k]).wait()

    _issue(ric_v.at[0, 0, 0], 0)
    _issue(ric_v.at[0, 1, 0], 1)

    def super_body(t, carry):
        for p in range(2):
            b = 2 * t + p

            for q in range(BLK):
                if q == BLK - 2:
                    @pl.when(b + 1 < NBLK)
                    def _():
                        nbs = pl.ds((b + 1) * BLK, BLK)
                        pltpu.make_async_copy(ric_hbm.at[wid, nbs],
                                              ric_v.at[1 - p], semb).wait()

                _drain(ric_v.at[p, q, 0], q % 2)
                pltpu.sync_copy(bufs[q % 2], acc.at[ric_v.at[p, q, 1]],
                                add=True)
                if q + 2 < BLK:
                    _issue(ric_v.at[p, q + 2, 0], q % 2)
                else:

                    @pl.when(b + 1 < NBLK)
                    def _():
                        _issue(ric_v.at[1 - p, q + 2 - BLK, 0], q % 2)
            @pl.when(b + 2 < NBLK)
            def _():
                n2bs = pl.ds((b + 2) * BLK, BLK)
                pltpu.async_copy(ric_hbm.at[wid, n2bs], ric_v.at[p], semb)
        return carry

    lax.fori_loop(0, NBLK // 2, super_body, 0, unroll=False)

    plsc.subcore_barrier()
    pltpu.sync_copy(acc.at[slab], out_hbm.at[cid].at[slab])


def _bn_relu(y, g, b):
    mean = jnp.mean(y, axis=0, keepdims=True)
    var = jnp.mean((y - mean) ** 2, axis=0, keepdims=True)
    return jnp.maximum((y - mean) * lax.rsqrt(var + 1e-5) * g + b, 0.0)


def _mm(a, w):
    return jnp.dot(a, w, preferred_element_type=jnp.float32)


def _prep_body(degacc, x, w10, b1, g1, be1, w20, dis_o, h1_o, xs1_o, y2_o):
    deg = degacc[0, :N, 0:1] + degacc[1, :N, 0:1]
    dis = jnp.where(deg > 0, lax.rsqrt(jnp.maximum(deg, 1e-12)), 0.0)
    h = _bn_relu(_mm(x[...], w10[...]) + b1[...], g1[...], be1[...])
    dis_o[...] = dis
    h1_o[...] = h
    xs1_o[...] = dis * h
    y2_o[...] = _mm(h, w20[...])


def _txs_body(S, dis, t_o, xs_o):
    t = -dis[...] * (S[0, :N, :] + S[1, :N, :])
    t_o[...] = t
    xs_o[...] = dis[...] * t


def _txs2_body(S, dis, tx0, t_o, xs_o):
    t = -2.0 * dis[...] * (S[0, :N, :] + S[1, :N, :]) - tx0[...]
    t_o[...] = t
    xs_o[...] = dis[...] * t


def _acc_body(y_in, t, wk, y_o):
    y_o[...] = y_in[...] + _mm(t[...], wk[...])


def _end_body(S, dis, tx0, wk, y_in, b2, g2, be2, wn0, h_o, xs_o, yn_o):
    t = -2.0 * dis[...] * (S[0, :N, :] + S[1, :N, :]) - tx0[...]
    y = y_in[...] + _mm(t, wk[...]) + b2[...]
    h = _bn_relu(y, g2[...], be2[...])
    h_o[...] = h
    xs_o[...] = dis[...] * h
    yn_o[...] = _mm(h, wn0[...])


def _final_body(S, dis, tx0, wk, y_in, b3, out_o):
    t = -2.0 * dis[...] * (S[0, :N, :] + S[1, :N, :]) - tx0[...]
    out_o[...] = y_in[...] + _mm(t, wk[...]) + b3[...]


_nd = jax.ShapeDtypeStruct((N, D), jnp.float32)
_tc_prep = pl.pallas_call(_prep_body, out_shape=(
    jax.ShapeDtypeStruct((N, 1), jnp.float32), _nd, _nd, _nd))
_tc_txs = pl.pallas_call(_txs_body, out_shape=(_nd, _nd))
_tc_txs2 = pl.pallas_call(_txs2_body, out_shape=(_nd, _nd))
_tc_acc = pl.pallas_call(_acc_body, out_shape=_nd)
_tc_end = pl.pallas_call(_end_body, out_shape=(_nd, _nd, _nd))
_tc_final = pl.pallas_call(_final_body, out_shape=_nd)


def kernel(x, edge_index, W1, b1, W2, b2, W3, b3, gamma1, beta1, gamma2,
           beta2):
    pad = EPWP - EPW
    padrows = jnp.broadcast_to(jnp.arange(pad, dtype=jnp.int32) % 128,
                               (NW, pad))
    row = jnp.concatenate(
        [edge_index[0].reshape(NW, EPW), padrows], axis=1).reshape(NW, NCH, C)
    col = jnp.pad(edge_index[1].reshape(NW, EPW), ((0, 0), (0, pad)),
                  constant_values=N).reshape(NW, NCH, C)
    zeros = jnp.zeros((NPAD, D), jnp.float32)
    ones = jnp.ones((C, D), jnp.float32)
    b1r = b1.reshape(1, D)
    b2r = b2.reshape(1, D)
    b3r = b3.reshape(1, D)
    g1 = gamma1.reshape(1, D)
    be1 = beta1.reshape(1, D)
    g2 = gamma2.reshape(1, D)
    be2 = beta2.reshape(1, D)

    degacc, colm = _deg_kernel(row, col, ones, zeros)

    dis, h1, xs1, y2a = _tc_prep(degacc, x, W1[0], b1r, g1, be1, W2[0])

    S1 = _prop_kernel(xs1, colm, zeros)
    t1, xs_t1 = _tc_txs(S1, dis)
    S2 = _prop_kernel(xs_t1, colm, zeros)
    y2b = _tc_acc(y2a, t1, W2[1])
    h2, xs2, y3a = _tc_end(S2, dis, h1, W2[2], y2b, b2r, g2, be2, W3[0])

    S3 = _prop_kernel(xs2, colm, zeros)
    u1, xs_u1 = _tc_txs(S3, dis)
    S4 = _prop_kernel(xs_u1, colm, zeros)
    y3b = _tc_acc(y3a, u1, W3[1])
    u2, xs_u2 = _tc_txs2(S4, dis, h2)
    S5 = _prop_kernel(xs_u2, colm, zeros)
    y3c = _tc_acc(y3b, u2, W3[2])
    u3, xs_u3 = _tc_txs2(S5, dis, u1)
    S6 = _prop_kernel(xs_u3, colm, zeros)
    y3d = _tc_acc(y3c, u3, W3[3])
    return _tc_final(S6, dis, u2, W3[4], y3d, b3r)
